# baseline (device time: 62387 ns/iter reference)
import jax
import jax.numpy as jnp
from jax import lax
from jax.experimental import pallas as pl
from jax.experimental.pallas import tpu as pltpu

N_Y = 4
N_P = 8
M_PER = 1024
R = M_PER // N_P
NW = 4
W = R // NW
D = 1024
EPS = 1e-6


def kernel(partial, gamma):
    part = partial.reshape(N_Y * M_PER, D)
    gamma2 = gamma.reshape(1, D)

    def body(part_ref, gamma_ref, out_ref,
             ybuf, ysend, yrecv, rsend, rrecv, lsend, lrecv):
        my_x = lax.axis_index("x")
        my_y = lax.axis_index("y")
        my_z = lax.axis_index("z")

        p = jnp.where(my_x == 0, my_z, 7 - my_z)

        def coords_of(pos):
            xx = pos // 4
            zz = jnp.where(xx == 0, pos % 4, 7 - pos)
            return xx, zz

        nxt = coords_of((p + 1) % N_P)
        prv = coords_of((p - 1) % N_P)
        nxt_dev = (nxt[0], my_y, nxt[1])
        prv_dev = (prv[0], my_y, prv[1])

        barrier_sem = pltpu.get_barrier_semaphore()
        for o in range(1, N_Y):
            pl.semaphore_signal(
                barrier_sem, inc=1,
                device_id=(my_x, (my_y + o) % N_Y, my_z),
                device_id_type=pl.DeviceIdType.MESH,
            )
        for dev in (nxt_dev, prv_dev):
            pl.semaphore_signal(
                barrier_sem, inc=1,
                device_id=dev, device_id_type=pl.DeviceIdType.MESH,
            )
        pl.semaphore_wait(barrier_sem, 5)

        y_sends = {}
        for w in range(NW):
            for o in range(1, N_Y):
                r = (my_y + o) % N_Y
                y_sends[(w, o)] = pltpu.make_async_remote_copy(
                    src_ref=part_ref.at[
                        pl.ds(r * M_PER + p * R + w * W, W), :],
                    dst_ref=ybuf.at[w, o - 1],
                    send_sem=ysend.at[w * 3 + o - 1],
                    recv_sem=yrecv.at[w * 3 + o - 1],
                    device_id=(my_x, r, my_z),
                    device_id_type=pl.DeviceIdType.MESH,
                )
            for o in range(1, N_Y):
                y_sends[(w, o)].start()
            if w < NW - 1:
                for o in range(1, N_Y):
                    y_sends[(w, o)].wait_send()

        def ring_step(w, s):
            sends, recvs = [], []
            if s < 3:
                q_out = (p - s) % N_P
                q_in = (p - 1 - s) % N_P
                sends.append(pltpu.make_async_remote_copy(
                    src_ref=out_ref.at[pl.ds(q_out * R + w * W, W), :],
                    dst_ref=out_ref.at[pl.ds(q_out * R + w * W, W), :],
                    send_sem=rsend.at[w * 3 + s],
                    recv_sem=rrecv.at[w * 3 + s],
                    device_id=nxt_dev,
                    device_id_type=pl.DeviceIdType.MESH,
                ))
                recvs.append(pltpu.make_async_remote_copy(
                    src_ref=out_ref.at[pl.ds(q_in * R + w * W, W), :],
                    dst_ref=out_ref.at[pl.ds(q_in * R + w * W, W), :],
                    send_sem=rsend.at[w * 3 + s],
                    recv_sem=rrecv.at[w * 3 + s],
                    device_id=prv_dev,
                    device_id_type=pl.DeviceIdType.MESH,
                ))
            q_out = (p + s) % N_P
            q_in = (p + 1 + s) % N_P
            sends.append(pltpu.make_async_remote_copy(
                src_ref=out_ref.at[pl.ds(q_out * R + w * W, W), :],
                dst_ref=out_ref.at[pl.ds(q_out * R + w * W, W), :],
                send_sem=lsend.at[w * 4 + s],
                recv_sem=lrecv.at[w * 4 + s],
                device_id=prv_dev,
                device_id_type=pl.DeviceIdType.MESH,
            ))
            recvs.append(pltpu.make_async_remote_copy(
                src_ref=out_ref.at[pl.ds(q_in * R + w * W, W), :],
                dst_ref=out_ref.at[pl.ds(q_in * R + w * W, W), :],
                send_sem=lsend.at[w * 4 + s],
                recv_sem=lrecv.at[w * 4 + s],
                device_id=nxt_dev,
                device_id_type=pl.DeviceIdType.MESH,
            ))
            return sends, recvs

        ring_sends = []
        ring_recvs = {}
        for t in range(NW + 4):
            for w in range(NW):
                s = t - w
                if s == 0:
                    for o in range(1, N_Y):
                        y_sends[(w, o)].wait_recv()
                    own = part_ref[
                        pl.ds(my_y * M_PER + p * R + w * W, W), :]
                    y_loc = own + ybuf[w, 0] + ybuf[w, 1] + ybuf[w, 2]
                    ms = jnp.mean(y_loc * y_loc, axis=-1, keepdims=True)
                    out_ref[pl.ds(p * R + w * W, W), :] = (
                        y_loc * lax.rsqrt(ms + EPS) * gamma_ref[...]
                    )
                elif 1 <= s <= 4:
                    for rd in ring_recvs[(w, s - 1)]:
                        rd.wait_recv()
                if 0 <= s <= 3:
                    sends, recvs = ring_step(w, s)
                    for rd in sends:
                        rd.start()
                    ring_sends += sends
                    ring_recvs[(w, s)] = recvs

        for o in range(1, N_Y):
            y_sends[(NW - 1, o)].wait_send()
        for rd in ring_sends:
            rd.wait_send()

    return pl.pallas_call(
        body,
        out_shape=jax.ShapeDtypeStruct((M_PER, D), jnp.float32),
        in_specs=[
            pl.BlockSpec(memory_space=pltpu.VMEM),
            pl.BlockSpec(memory_space=pltpu.VMEM),
        ],
        out_specs=pl.BlockSpec(memory_space=pltpu.VMEM),
        scratch_shapes=[
            pltpu.VMEM((NW, N_Y - 1, W, D), jnp.float32),
            pltpu.SemaphoreType.DMA((NW * 3,)),
            pltpu.SemaphoreType.DMA((NW * 3,)),
            pltpu.SemaphoreType.DMA((NW * 3,)),
            pltpu.SemaphoreType.DMA((NW * 3,)),
            pltpu.SemaphoreType.DMA((NW * 4,)),
            pltpu.SemaphoreType.DMA((NW * 4,)),
        ],
        compiler_params=pltpu.CompilerParams(collective_id=0),
    )(part, gamma2)


# device time: 57244 ns/iter; 1.0898x vs baseline; 1.0898x over previous
import jax
import jax.numpy as jnp
from jax import lax
from jax.experimental import pallas as pl
from jax.experimental.pallas import tpu as pltpu

N_Y = 4
N_P = 8
M_PER = 1024
R = M_PER // N_P
W = R // 2
D = 1024
EPS = 1e-6


def kernel(partial, gamma):
    part = partial.reshape(N_Y * M_PER, D)
    gamma2 = gamma.reshape(1, D)

    def body(part_ref, gamma_ref, out_ref,
             ybuf, ysend, yrecv, rsend, rrecv, lsend, lrecv):
        my_x = lax.axis_index("x")
        my_y = lax.axis_index("y")
        my_z = lax.axis_index("z")

        p = jnp.where(my_x == 0, my_z, 7 - my_z)

        def coords_of(pos):
            xx = pos // 4
            zz = jnp.where(xx == 0, pos % 4, 7 - pos)
            return xx, zz

        nxt_x, nxt_z = coords_of((p + 1) % N_P)
        prv_x, prv_z = coords_of((p - 1) % N_P)

        barrier_sem = pltpu.get_barrier_semaphore()
        for o in range(1, N_Y):
            pl.semaphore_signal(
                barrier_sem, inc=1,
                device_id=(my_x, (my_y + o) % N_Y, my_z),
                device_id_type=pl.DeviceIdType.MESH,
            )
        for dev in ((nxt_x, my_y, nxt_z), (prv_x, my_y, prv_z)):
            pl.semaphore_signal(
                barrier_sem, inc=1,
                device_id=dev, device_id_type=pl.DeviceIdType.MESH,
            )
        pl.semaphore_wait(barrier_sem, 5)

        y_sends = {}
        for w in (0, 1):
            for o in range(1, N_Y):
                r = (my_y + o) % N_Y
                rd = pltpu.make_async_remote_copy(
                    src_ref=part_ref.at[
                        pl.ds(r * M_PER + p * R + w * W, W), :],
                    dst_ref=ybuf.at[w, o - 1],
                    send_sem=ysend.at[w * 3 + o - 1],
                    recv_sem=yrecv.at[w * 3 + o - 1],
                    device_id=(my_x, r, my_z),
                    device_id_type=pl.DeviceIdType.MESH,
                )
                y_sends[(w, o)] = rd
            if w == 0:
                for o in range(1, N_Y):
                    y_sends[(0, o)].start()
                for o in range(1, N_Y):
                    y_sends[(0, o)].wait_send()
            else:
                for o in range(1, N_Y):
                    y_sends[(1, o)].start()

        def ring_step(w, s):
            sends, recvs = [], []
            base = p * R + 0
            if s < 3:
                q_out = (p - s) % N_P
                q_in = (p - 1 - s) % N_P
                sends.append(pltpu.make_async_remote_copy(
                    src_ref=out_ref.at[pl.ds(q_out * R + w * W, W), :],
                    dst_ref=out_ref.at[pl.ds(q_out * R + w * W, W), :],
                    send_sem=rsend.at[w * 3 + s],
                    recv_sem=rrecv.at[w * 3 + s],
                    device_id=(nxt_x, my_y, nxt_z),
                    device_id_type=pl.DeviceIdType.MESH,
                ))
                recvs.append(pltpu.make_async_remote_copy(
                    src_ref=out_ref.at[pl.ds(q_in * R + w * W, W), :],
                    dst_ref=out_ref.at[pl.ds(q_in * R + w * W, W), :],
                    send_sem=rsend.at[w * 3 + s],
                    recv_sem=rrecv.at[w * 3 + s],
                    device_id=(prv_x, my_y, prv_z),
                    device_id_type=pl.DeviceIdType.MESH,
                ))
            q_out = (p + s) % N_P
            q_in = (p + 1 + s) % N_P
            sends.append(pltpu.make_async_remote_copy(
                src_ref=out_ref.at[pl.ds(q_out * R + w * W, W), :],
                dst_ref=out_ref.at[pl.ds(q_out * R + w * W, W), :],
                send_sem=lsend.at[w * 4 + s],
                recv_sem=lrecv.at[w * 4 + s],
                device_id=(prv_x, my_y, prv_z),
                device_id_type=pl.DeviceIdType.MESH,
            ))
            recvs.append(pltpu.make_async_remote_copy(
                src_ref=out_ref.at[pl.ds(q_in * R + w * W, W), :],
                dst_ref=out_ref.at[pl.ds(q_in * R + w * W, W), :],
                send_sem=lsend.at[w * 4 + s],
                recv_sem=lrecv.at[w * 4 + s],
                device_id=(nxt_x, my_y, nxt_z),
                device_id_type=pl.DeviceIdType.MESH,
            ))
            return sends, recvs

        ring_sends = []
        ring_recvs = {}
        for w in (0, 1):
            for o in range(1, N_Y):
                y_sends[(w, o)].wait_recv()
            own = part_ref[pl.ds(my_y * M_PER + p * R + w * W, W), :]
            y_loc = own + ybuf[w, 0] + ybuf[w, 1] + ybuf[w, 2]
            ms = jnp.mean(y_loc * y_loc, axis=-1, keepdims=True)
            out_ref[pl.ds(p * R + w * W, W), :] = (
                y_loc * lax.rsqrt(ms + EPS) * gamma_ref[...]
            )
            sends, recvs = ring_step(w, 0)
            for rd in sends:
                rd.start()
            ring_sends += sends
            ring_recvs[(w, 0)] = recvs
        for s in range(1, 4):
            for w in (0, 1):
                for rd in ring_recvs[(w, s - 1)]:
                    rd.wait_recv()
                sends, recvs = ring_step(w, s)
                for rd in sends:
                    rd.start()
                ring_sends += sends
                ring_recvs[(w, s)] = recvs
        for w in (0, 1):
            for rd in ring_recvs[(w, 3)]:
                rd.wait_recv()

        for o in range(1, N_Y):
            y_sends[(1, o)].wait_send()
        for rd in ring_sends:
            rd.wait_send()

    return pl.pallas_call(
        body,
        out_shape=jax.ShapeDtypeStruct((M_PER, D), jnp.float32),
        in_specs=[
            pl.BlockSpec(memory_space=pltpu.VMEM),
            pl.BlockSpec(memory_space=pltpu.VMEM),
        ],
        out_specs=pl.BlockSpec(memory_space=pltpu.VMEM),
        scratch_shapes=[
            pltpu.VMEM((2, N_Y - 1, W, D), jnp.float32),
            pltpu.SemaphoreType.DMA((6,)),
            pltpu.SemaphoreType.DMA((6,)),
            pltpu.SemaphoreType.DMA((6,)),
            pltpu.SemaphoreType.DMA((6,)),
            pltpu.SemaphoreType.DMA((8,)),
            pltpu.SemaphoreType.DMA((8,)),
        ],
        compiler_params=pltpu.CompilerParams(collective_id=0),
    )(part, gamma2)
